# baseline (device time: 33391 ns/iter reference)
import jax
import jax.numpy as jnp
from jax import lax
from jax.experimental import pallas as pl
from jax.experimental.pallas import tpu as pltpu

N_DEV = 4
B, SQ, SKV, D = 2, 128, 128, 512
H, DH = 8, 64
SCALE = 0.125


def kernel(x, Wq, Wo, K_ext, V_ext):
    K = jnp.transpose(K_ext, (0, 2, 1, 3))
    V = jnp.transpose(V_ext, (0, 2, 1, 3))

    def body(x_ref, wq_ref, wo_ref, k_ref, v_ref, out_ref,
             comm_ref, send_sems, recv_sems):
        my = lax.axis_index("i")
        left = lax.rem(my + N_DEV - 1, N_DEV)
        right = lax.rem(my + 1, N_DEV)

        barrier = pltpu.get_barrier_semaphore()
        for nbr in (left, right):
            pl.semaphore_signal(barrier, inc=1, device_id=(nbr,),
                                device_id_type=pl.DeviceIdType.MESH)
        pl.semaphore_wait(barrier, 2)

        for b in range(B):
            qb = jnp.dot(x_ref[b], wq_ref[...],
                         preferred_element_type=jnp.float32)
            outs = []
            for h in range(H):
                qh = qb[:, h * DH:(h + 1) * DH]
                kh = k_ref[b, h]
                vh = v_ref[b, h]
                s = jnp.dot(qh, kh.T,
                            preferred_element_type=jnp.float32) * SCALE
                m = jnp.max(s, axis=1, keepdims=True)
                p = jnp.exp(s - m)
                l = jnp.sum(p, axis=1, keepdims=True)
                outs.append(
                    jnp.dot(p, vh, preferred_element_type=jnp.float32) / l)
            attn = jnp.concatenate(outs, axis=1)
            pb = jnp.dot(attn, wo_ref[...],
                         preferred_element_type=jnp.float32)
            out_ref[b] = pb
            comm_ref[0, b] = pb

        for hop in range(N_DEV - 1):
            rdma = pltpu.make_async_remote_copy(
                src_ref=comm_ref.at[hop],
                dst_ref=comm_ref.at[hop + 1],
                send_sem=send_sems.at[hop],
                recv_sem=recv_sems.at[hop],
                device_id=(right,),
                device_id_type=pl.DeviceIdType.MESH,
            )
            rdma.start()
            rdma.wait()
            out_ref[...] += comm_ref[hop + 1]

    return pl.pallas_call(
        body,
        out_shape=jax.ShapeDtypeStruct((B, SQ, D), jnp.float32),
        in_specs=[pl.BlockSpec(memory_space=pltpu.VMEM)] * 5,
        out_specs=pl.BlockSpec(memory_space=pltpu.VMEM),
        scratch_shapes=[
            pltpu.VMEM((N_DEV, B, SQ, D), jnp.float32),
            pltpu.SemaphoreType.DMA((N_DEV - 1,)),
            pltpu.SemaphoreType.DMA((N_DEV - 1,)),
        ],
        compiler_params=pltpu.CompilerParams(collective_id=0),
    )(x, Wq, Wo, K, V)


# device time: 26070 ns/iter; 1.2808x vs baseline; 1.2808x over previous
import jax
import jax.numpy as jnp
from jax import lax
from jax.experimental import pallas as pl
from jax.experimental.pallas import tpu as pltpu

N_DEV = 4
B, SQ, SKV, D = 2, 128, 128, 512
H_HEADS, DH = 8, 64
SCALE = 0.125
R = B * SQ


def kernel(x, Wq, Wo, K_ext, V_ext):
    K = jnp.transpose(K_ext, (0, 2, 1, 3))
    V = jnp.transpose(V_ext, (0, 2, 1, 3))

    def body(x_ref, wq_ref, wo_ref, k_ref, v_ref, out_ref,
             recv1_ref, recv2_ref, send_sems, recv_sems):
        my = lax.axis_index("i")
        p1 = my ^ 1
        p2 = my ^ 3
        half = (my ^ (my >> 1)) & 1
        quart = (my >> 1) & 1

        barrier = pltpu.get_barrier_semaphore()
        for nbr in (p1, p2):
            pl.semaphore_signal(barrier, inc=1, device_id=(nbr,),
                                device_id_type=pl.DeviceIdType.MESH)
        pl.semaphore_wait(barrier, 2)

        for b in range(B):
            qb = jnp.dot(x_ref[b], wq_ref[...],
                         preferred_element_type=jnp.float32)
            outs = []
            for h in range(H_HEADS):
                qh = qb[:, h * DH:(h + 1) * DH]
                kh = k_ref[b, h]
                vh = v_ref[b, h]
                s = jnp.dot(qh, kh.T,
                            preferred_element_type=jnp.float32) * SCALE
                m = jnp.max(s, axis=1, keepdims=True)
                p = jnp.exp(s - m)
                l = jnp.sum(p, axis=1, keepdims=True)
                outs.append(
                    jnp.dot(p, vh, preferred_element_type=jnp.float32) / l)
            attn = jnp.concatenate(outs, axis=1)
            out_ref[pl.ds(b * SQ, SQ)] = jnp.dot(
                attn, wo_ref[...], preferred_element_type=jnp.float32)

        hs = half * (R // 2)
        gs = (1 - half) * (R // 2)
        qs = hs + quart * (R // 4)
        gq = hs + (1 - quart) * (R // 4)

        rd1 = pltpu.make_async_remote_copy(
            src_ref=out_ref.at[pl.ds(gs, R // 2)],
            dst_ref=recv1_ref,
            send_sem=send_sems.at[0], recv_sem=recv_sems.at[0],
            device_id=(p1,), device_id_type=pl.DeviceIdType.MESH,
        )
        rd1.start()
        rd1.wait()
        out_ref[pl.ds(hs, R // 2)] = out_ref[pl.ds(hs, R // 2)] + recv1_ref[...]

        rd2 = pltpu.make_async_remote_copy(
            src_ref=out_ref.at[pl.ds(gq, R // 4)],
            dst_ref=recv2_ref,
            send_sem=send_sems.at[1], recv_sem=recv_sems.at[1],
            device_id=(p2,), device_id_type=pl.DeviceIdType.MESH,
        )
        rd2.start()
        rd2.wait()
        out_ref[pl.ds(qs, R // 4)] = out_ref[pl.ds(qs, R // 4)] + recv2_ref[...]

        rd3 = pltpu.make_async_remote_copy(
            src_ref=out_ref.at[pl.ds(qs, R // 4)],
            dst_ref=out_ref.at[pl.ds(qs, R // 4)],
            send_sem=send_sems.at[2], recv_sem=recv_sems.at[2],
            device_id=(p2,), device_id_type=pl.DeviceIdType.MESH,
        )
        rd3.start()
        rd3.wait()

        rd4 = pltpu.make_async_remote_copy(
            src_ref=out_ref.at[pl.ds(hs, R // 2)],
            dst_ref=out_ref.at[pl.ds(hs, R // 2)],
            send_sem=send_sems.at[3], recv_sem=recv_sems.at[3],
            device_id=(p1,), device_id_type=pl.DeviceIdType.MESH,
        )
        rd4.start()
        rd4.wait()

    flat = pl.pallas_call(
        body,
        out_shape=jax.ShapeDtypeStruct((R, D), jnp.float32),
        in_specs=[pl.BlockSpec(memory_space=pltpu.VMEM)] * 5,
        out_specs=pl.BlockSpec(memory_space=pltpu.VMEM),
        scratch_shapes=[
            pltpu.VMEM((R // 2, D), jnp.float32),
            pltpu.VMEM((R // 4, D), jnp.float32),
            pltpu.SemaphoreType.DMA((4,)),
            pltpu.SemaphoreType.DMA((4,)),
        ],
        compiler_params=pltpu.CompilerParams(collective_id=0),
    )(x, Wq, Wo, K, V)
    return flat.reshape(B, SQ, D)


# device time: 23695 ns/iter; 1.4092x vs baseline; 1.1002x over previous
import jax
import jax.numpy as jnp
from jax import lax
from jax.experimental import pallas as pl
from jax.experimental.pallas import tpu as pltpu

N_DEV = 4
B, SQ, SKV, D = 2, 128, 128, 512
H_HEADS, DH = 8, 64
SCALE = 0.125
R = B * SQ
HALF = R // 2


def kernel(x, Wq, Wo, K_ext, V_ext):
    xf = x.reshape(R, D).astype(jnp.bfloat16)
    K = jnp.transpose(K_ext, (0, 2, 1, 3)).reshape(B * H_HEADS, SKV, DH)
    K = K.astype(jnp.bfloat16)
    V = jnp.transpose(V_ext, (0, 2, 1, 3)).reshape(B * H_HEADS, SKV, DH)
    V = V.astype(jnp.bfloat16)
    Wqb = Wq.astype(jnp.bfloat16)
    Wob = Wo.astype(jnp.bfloat16)

    def body(x_ref, wq_ref, wo_ref, k_ref, v_ref, out_ref,
             recv1_ref, recv2_ref, send_sems, recv_sems):
        my = lax.axis_index("i")
        p1 = my ^ 1
        p2 = my ^ 3
        half = (my ^ (my >> 1)) & 1

        barrier = pltpu.get_barrier_semaphore()
        for nbr in (p1, p2):
            pl.semaphore_signal(barrier, inc=1, device_id=(nbr,),
                                device_id_type=pl.DeviceIdType.MESH,)
        pl.semaphore_wait(barrier, 2)

        wqb = wq_ref[...]
        wob = wo_ref[...]

        def compute_partial(b):
            xb = x_ref[pl.ds(b * SQ, SQ)]
            qb = lax.dot_general(
                xb, wqb, (((1,), (0,)), ((), ())),
                preferred_element_type=jnp.float32)
            qb = qb.astype(jnp.bfloat16)
            outs = []
            for h in range(H_HEADS):
                kh = k_ref[pl.ds(b * H_HEADS + h, 1)][0]
                vh = v_ref[pl.ds(b * H_HEADS + h, 1)][0]
                qh = qb[:, h * DH:(h + 1) * DH]
                s = lax.dot_general(
                    qh, kh, (((1,), (1,)), ((), ())),
                    preferred_element_type=jnp.float32) * SCALE
                m = jnp.max(s, axis=1, keepdims=True)
                p = jnp.exp(s - m)
                l = jnp.sum(p, axis=1, keepdims=True)
                pv = lax.dot_general(
                    p.astype(jnp.bfloat16), vh, (((1,), (0,)), ((), ())),
                    preferred_element_type=jnp.float32)
                outs.append(pv / l)
            attn = jnp.concatenate(outs, axis=1).astype(jnp.bfloat16)
            return lax.dot_general(
                attn, wob, (((1,), (0,)), ((), ())),
                preferred_element_type=jnp.float32)

        hs = half * HALF
        gs = (1 - half) * HALF

        out_ref[pl.ds(gs, HALF)] = compute_partial(1 - half)
        rd1 = pltpu.make_async_remote_copy(
            src_ref=out_ref.at[pl.ds(gs, HALF)],
            dst_ref=recv1_ref,
            send_sem=send_sems.at[0], recv_sem=recv_sems.at[0],
            device_id=(p1,), device_id_type=pl.DeviceIdType.MESH,
        )
        rd1.start()
        out_ref[pl.ds(hs, HALF)] = compute_partial(half)
        rd1.wait()
        out_ref[pl.ds(hs, HALF)] = out_ref[pl.ds(hs, HALF)] + recv1_ref[...]

        rd2 = pltpu.make_async_remote_copy(
            src_ref=out_ref.at[pl.ds(hs, HALF)],
            dst_ref=recv2_ref,
            send_sem=send_sems.at[1], recv_sem=recv_sems.at[1],
            device_id=(p2,), device_id_type=pl.DeviceIdType.MESH,
        )
        rd2.start()
        rd2.wait()
        out_ref[pl.ds(hs, HALF)] = out_ref[pl.ds(hs, HALF)] + recv2_ref[...]

        rd3 = pltpu.make_async_remote_copy(
            src_ref=out_ref.at[pl.ds(hs, HALF)],
            dst_ref=out_ref.at[pl.ds(hs, HALF)],
            send_sem=send_sems.at[2], recv_sem=recv_sems.at[2],
            device_id=(p1,), device_id_type=pl.DeviceIdType.MESH,
        )
        rd3.start()
        rd3.wait()

    flat = pl.pallas_call(
        body,
        out_shape=jax.ShapeDtypeStruct((R, D), jnp.float32),
        in_specs=[pl.BlockSpec(memory_space=pltpu.VMEM)] * 5,
        out_specs=pl.BlockSpec(memory_space=pltpu.VMEM),
        scratch_shapes=[
            pltpu.VMEM((HALF, D), jnp.float32),
            pltpu.VMEM((HALF, D), jnp.float32),
            pltpu.SemaphoreType.DMA((3,)),
            pltpu.SemaphoreType.DMA((3,)),
        ],
        compiler_params=pltpu.CompilerParams(collective_id=0),
    )(xf, Wqb, Wob, K, V)
    return flat.reshape(B, SQ, D)


# device time: 22295 ns/iter; 1.4977x vs baseline; 1.0628x over previous
import jax
import jax.numpy as jnp
from jax import lax
from jax.experimental import pallas as pl
from jax.experimental.pallas import tpu as pltpu

N_DEV = 4
B, SQ, SKV, D = 2, 128, 128, 512
H_HEADS, DH = 8, 64
SCALE = 0.125
R = B * SQ
HALF = R // 2
CHALF = D // 2


def kernel(x, Wq, Wo, K_ext, V_ext):
    xf = x.reshape(R, D)
    Kf = K_ext.reshape(B, SKV, H_HEADS * DH)
    Vf = V_ext.reshape(B, SKV, H_HEADS * DH)

    def body(x_ref, wq_ref, wo_ref, k_ref, v_ref, out_ref,
             recvA1, recvA2, recvB1, recvB2, send_sems, recv_sems):
        my = lax.axis_index("i")
        p1 = my ^ 1
        p2 = my ^ 3
        ha = (my ^ (my >> 1)) & 1
        hb = (my >> 1) & 1

        barrier = pltpu.get_barrier_semaphore()
        for nbr in (p1, p2):
            pl.semaphore_signal(barrier, inc=1, device_id=(nbr,),
                                device_id_type=pl.DeviceIdType.MESH,)
        pl.semaphore_wait(barrier, 2)

        wqb = wq_ref[...].astype(jnp.bfloat16)
        wob = wo_ref[...].astype(jnp.bfloat16)

        def compute_partial(b):
            xb = x_ref[pl.ds(b * SQ, SQ)].astype(jnp.bfloat16)
            kb = k_ref[pl.ds(b, 1)][0].astype(jnp.bfloat16)
            vb = v_ref[pl.ds(b, 1)][0].astype(jnp.bfloat16)
            qb = lax.dot_general(
                xb, wqb, (((1,), (0,)), ((), ())),
                preferred_element_type=jnp.float32)
            qb = qb.astype(jnp.bfloat16)
            outs = []
            for h in range(H_HEADS):
                sl = slice(h * DH, (h + 1) * DH)
                qh, kh, vh = qb[:, sl], kb[:, sl], vb[:, sl]
                s = lax.dot_general(
                    qh, kh, (((1,), (1,)), ((), ())),
                    preferred_element_type=jnp.float32) * SCALE
                m = jnp.max(s, axis=1, keepdims=True)
                p = jnp.exp(s - m)
                l = jnp.sum(p, axis=1, keepdims=True)
                pv = lax.dot_general(
                    p.astype(jnp.bfloat16), vh, (((1,), (0,)), ((), ())),
                    preferred_element_type=jnp.float32)
                outs.append(pv * (1.0 / l))
            attn = jnp.concatenate(outs, axis=1).astype(jnp.bfloat16)
            return lax.dot_general(
                attn, wob, (((1,), (0,)), ((), ())),
                preferred_element_type=jnp.float32)

        CA = pl.ds(0, CHALF)
        CB = pl.ds(CHALF, CHALF)

        def exchange(rows, cols, dst, sem, dev):
            return pltpu.make_async_remote_copy(
                src_ref=out_ref.at[pl.ds(rows, HALF), cols],
                dst_ref=dst,
                send_sem=send_sems.at[sem], recv_sem=recv_sems.at[sem],
                device_id=(dev,), device_id_type=pl.DeviceIdType.MESH,
            )

        def add(rows, cols, recv):
            out_ref[pl.ds(rows, HALF), cols] = (
                out_ref[pl.ds(rows, HALF), cols] + recv[...])

        out_ref[pl.ds((1 - ha) * HALF, HALF)] = compute_partial(1 - ha)
        a1 = exchange((1 - ha) * HALF, CA, recvA1, 0, p1)
        a1.start()
        out_ref[pl.ds(ha * HALF, HALF)] = compute_partial(ha)
        b1 = exchange((1 - hb) * HALF, CB, recvB1, 1, p2)
        b1.start()

        a1.wait()
        add(ha * HALF, CA, recvA1)
        a2 = exchange(ha * HALF, CA, recvA2, 2, p2)
        a2.start()

        b1.wait()
        add(hb * HALF, CB, recvB1)
        b2 = exchange(hb * HALF, CB, recvB2, 3, p1)
        b2.start()

        a2.wait()
        add(ha * HALF, CA, recvA2)
        a3 = exchange(ha * HALF, CA,
                      out_ref.at[pl.ds(ha * HALF, HALF), CA], 4, p1)
        a3.start()

        b2.wait()
        add(hb * HALF, CB, recvB2)
        b3 = exchange(hb * HALF, CB,
                      out_ref.at[pl.ds(hb * HALF, HALF), CB], 5, p2)
        b3.start()

        a3.wait()
        b3.wait()

    flat = pl.pallas_call(
        body,
        out_shape=jax.ShapeDtypeStruct((R, D), jnp.float32),
        in_specs=[pl.BlockSpec(memory_space=pltpu.VMEM)] * 5,
        out_specs=pl.BlockSpec(memory_space=pltpu.VMEM),
        scratch_shapes=[
            pltpu.VMEM((HALF, CHALF), jnp.float32),
            pltpu.VMEM((HALF, CHALF), jnp.float32),
            pltpu.VMEM((HALF, CHALF), jnp.float32),
            pltpu.VMEM((HALF, CHALF), jnp.float32),
            pltpu.SemaphoreType.DMA((6,)),
            pltpu.SemaphoreType.DMA((6,)),
        ],
        compiler_params=pltpu.CompilerParams(collective_id=0),
    )(xf, Wq, Wo, Kf, Vf)
    return flat.reshape(B, SQ, D)


# device time: 19377 ns/iter; 1.7232x vs baseline; 1.1506x over previous
import jax
import jax.numpy as jnp
from jax import lax
from jax.experimental import pallas as pl
from jax.experimental.pallas import tpu as pltpu

N_DEV = 4
B, SQ, SKV, D = 2, 128, 128, 512
H_HEADS, DH = 8, 64
SCALE = 0.125
CHALF = D // 2


def kernel(x, Wq, Wo, K_ext, V_ext):
    def body(x_ref, wq_ref, wo_ref, k_ref, v_ref, out_ref,
             acc_ref, recvA1, recvA2, recvB1, recvB2, send_sems, recv_sems):
        my = lax.axis_index("i")
        p1 = my ^ 1
        p2 = my ^ 3
        ha = (my ^ (my >> 1)) & 1
        hb = (my >> 1) & 1

        wqb = wq_ref[...].astype(jnp.bfloat16)
        wob = wo_ref[...].astype(jnp.bfloat16)

        def compute_partial(b):
            xb = x_ref[pl.ds(b, 1)].reshape(SQ, D).astype(jnp.bfloat16)
            qb = lax.dot_general(
                xb, wqb, (((1,), (0,)), ((), ())),
                preferred_element_type=jnp.float32)
            qb = qb.astype(jnp.bfloat16)
            outs = []
            for h in range(H_HEADS):
                qh = qb[:, h * DH:(h + 1) * DH]
                kh = k_ref[pl.ds(b, 1), :, h, :].reshape(
                    SKV, DH).astype(jnp.bfloat16)
                vh = v_ref[pl.ds(b, 1), :, h, :].reshape(
                    SKV, DH).astype(jnp.bfloat16)
                s = lax.dot_general(
                    qh, kh, (((1,), (1,)), ((), ())),
                    preferred_element_type=jnp.float32) * SCALE
                m = jnp.max(s, axis=1, keepdims=True)
                p = jnp.exp(s - m)
                l = jnp.sum(p, axis=1, keepdims=True)
                pv = lax.dot_general(
                    p.astype(jnp.bfloat16), vh, (((1,), (0,)), ((), ())),
                    preferred_element_type=jnp.float32)
                outs.append(pv * (1.0 / l))
            attn = jnp.concatenate(outs, axis=1).astype(jnp.bfloat16)
            return lax.dot_general(
                attn, wob, (((1,), (0,)), ((), ())),
                preferred_element_type=jnp.float32,
            ).astype(jnp.bfloat16).reshape(1, SQ, D)

        CA = pl.ds(0, CHALF)
        CB = pl.ds(CHALF, CHALF)

        def exchange(b, cols, dst, sem, dev):
            return pltpu.make_async_remote_copy(
                src_ref=acc_ref.at[pl.ds(b, 1), :, cols],
                dst_ref=dst,
                send_sem=send_sems.at[sem], recv_sem=recv_sems.at[sem],
                device_id=(dev,), device_id_type=pl.DeviceIdType.MESH,
            )

        def add(b, cols, recv):
            acc_ref[pl.ds(b, 1), :, cols] = (
                acc_ref[pl.ds(b, 1), :, cols].astype(jnp.float32)
                + recv[...].astype(jnp.float32)
            ).astype(jnp.bfloat16)

        acc_ref[pl.ds(1 - ha, 1)] = compute_partial(1 - ha)
        barrier = pltpu.get_barrier_semaphore()
        for nbr in (p1, p2):
            pl.semaphore_signal(barrier, inc=1, device_id=(nbr,),
                                device_id_type=pl.DeviceIdType.MESH,)
        pl.semaphore_wait(barrier, 2)

        a1 = exchange(1 - ha, CA, recvA1, 0, p1)
        a1.start()
        acc_ref[pl.ds(ha, 1)] = compute_partial(ha)
        b1 = exchange(1 - hb, CB, recvB1, 1, p2)
        b1.start()

        a1.wait()
        add(ha, CA, recvA1)
        a2 = exchange(ha, CA, recvA2, 2, p2)
        a2.start()

        b1.wait()
        add(hb, CB, recvB1)
        b2 = exchange(hb, CB, recvB2, 3, p1)
        b2.start()

        a2.wait()
        add(ha, CA, recvA2)
        a3 = exchange(ha, CA, acc_ref.at[pl.ds(ha, 1), :, CA], 4, p1)
        a3.start()

        b2.wait()
        add(hb, CB, recvB2)
        b3 = exchange(hb, CB, acc_ref.at[pl.ds(hb, 1), :, CB], 5, p2)
        b3.start()

        a3.wait()
        b3.wait()
        out_ref[...] = acc_ref[...].astype(jnp.float32)

    return pl.pallas_call(
        body,
        out_shape=jax.ShapeDtypeStruct((B, SQ, D), jnp.float32),
        in_specs=[pl.BlockSpec(memory_space=pltpu.VMEM)] * 5,
        out_specs=pl.BlockSpec(memory_space=pltpu.VMEM),
        scratch_shapes=[
            pltpu.VMEM((B, SQ, D), jnp.bfloat16),
            pltpu.VMEM((1, SQ, CHALF), jnp.bfloat16),
            pltpu.VMEM((1, SQ, CHALF), jnp.bfloat16),
            pltpu.VMEM((1, SQ, CHALF), jnp.bfloat16),
            pltpu.VMEM((1, SQ, CHALF), jnp.bfloat16),
            pltpu.SemaphoreType.DMA((6,)),
            pltpu.SemaphoreType.DMA((6,)),
        ],
        compiler_params=pltpu.CompilerParams(collective_id=0),
    )(x, Wq, Wo, K_ext, V_ext)


# device time: 19371 ns/iter; 1.7238x vs baseline; 1.0003x over previous
import jax
import jax.numpy as jnp
from jax import lax
from jax.experimental import pallas as pl
from jax.experimental.pallas import tpu as pltpu

N_DEV = 4
B, SQ, SKV, D = 2, 128, 128, 512
H_HEADS, DH = 8, 64
SCALE = 0.125
CHALF = D // 2


def kernel(x, Wq, Wo, K_ext, V_ext):
    def body(x_ref, wq_ref, wo_ref, k_ref, v_ref, out_ref,
             acc_ref, recvA1, recvA2, recvB1, recvB2, send_sems, recv_sems):
        my = lax.axis_index("i")
        p1 = my ^ 1
        p2 = my ^ 3
        ha = (my ^ (my >> 1)) & 1
        hb = (my >> 1) & 1

        wqb = wq_ref[...].astype(jnp.bfloat16)
        wob = wo_ref[...].astype(jnp.bfloat16)

        def compute_partial(b):
            xb = x_ref[pl.ds(b, 1)].reshape(SQ, D).astype(jnp.bfloat16)
            qb = lax.dot_general(
                xb, wqb, (((1,), (0,)), ((), ())),
                preferred_element_type=jnp.float32)
            qb = qb.astype(jnp.bfloat16)
            outs = []
            for h in range(H_HEADS):
                qh = qb[:, h * DH:(h + 1) * DH]
                kh = k_ref[pl.ds(b, 1), :, h, :].reshape(
                    SKV, DH).astype(jnp.bfloat16)
                vh = v_ref[pl.ds(b, 1), :, h, :].reshape(
                    SKV, DH).astype(jnp.bfloat16)
                s = lax.dot_general(
                    qh, kh, (((1,), (1,)), ((), ())),
                    preferred_element_type=jnp.float32) * SCALE
                m = jnp.max(s, axis=1, keepdims=True)
                p = jnp.exp(s - m)
                l = jnp.sum(p, axis=1, keepdims=True)
                pv = lax.dot_general(
                    p.astype(jnp.bfloat16), vh, (((1,), (0,)), ((), ())),
                    preferred_element_type=jnp.float32)
                outs.append(pv * (1.0 / l))
            attn = jnp.concatenate(outs, axis=1).astype(jnp.bfloat16)
            return lax.dot_general(
                attn, wob, (((1,), (0,)), ((), ())),
                preferred_element_type=jnp.float32,
            ).astype(jnp.bfloat16).reshape(1, SQ, D)

        CA = pl.ds(0, CHALF)
        CB = pl.ds(CHALF, CHALF)

        def exchange(b, cols, dst, sem, dev):
            return pltpu.make_async_remote_copy(
                src_ref=acc_ref.at[pl.ds(b, 1), :, cols],
                dst_ref=dst,
                send_sem=send_sems.at[sem], recv_sem=recv_sems.at[sem],
                device_id=(dev,), device_id_type=pl.DeviceIdType.MESH,
            )

        def add(b, cols, recv):
            acc_ref[pl.ds(b, 1), :, cols] = (
                acc_ref[pl.ds(b, 1), :, cols].astype(jnp.float32)
                + recv[...].astype(jnp.float32)
            ).astype(jnp.bfloat16)

        acc_ref[pl.ds(1 - ha, 1)] = compute_partial(1 - ha)
        barrier = pltpu.get_barrier_semaphore()
        for nbr in (p1, p2):
            pl.semaphore_signal(barrier, inc=1, device_id=(nbr,),
                                device_id_type=pl.DeviceIdType.MESH,)
        pl.semaphore_wait(barrier, 2)

        a1 = exchange(1 - ha, CA, recvA1, 0, p1)
        a1.start()
        acc_ref[pl.ds(ha, 1)] = compute_partial(ha)
        b1 = exchange(1 - hb, CB, recvB1, 1, p2)
        b1.start()

        a1.wait()
        add(ha, CA, recvA1)
        a2 = exchange(ha, CA, recvA2, 2, p2)
        a2.start()

        b1.wait()
        add(hb, CB, recvB1)
        b2 = exchange(hb, CB, recvB2, 3, p1)
        b2.start()

        def emit(b, cols):
            out_ref[pl.ds(b, 1), :, cols] = (
                acc_ref[pl.ds(b, 1), :, cols].astype(jnp.float32))

        a2.wait()
        add(ha, CA, recvA2)
        a3 = exchange(ha, CA, acc_ref.at[pl.ds(ha, 1), :, CA], 4, p1)
        a3.start()
        emit(ha, CA)

        b2.wait()
        add(hb, CB, recvB2)
        b3 = exchange(hb, CB, acc_ref.at[pl.ds(hb, 1), :, CB], 5, p2)
        b3.start()
        emit(hb, CB)

        a3.wait()
        emit(1 - ha, CA)
        b3.wait()
        emit(1 - hb, CB)

    return pl.pallas_call(
        body,
        out_shape=jax.ShapeDtypeStruct((B, SQ, D), jnp.float32),
        in_specs=[pl.BlockSpec(memory_space=pltpu.VMEM)] * 5,
        out_specs=pl.BlockSpec(memory_space=pltpu.VMEM),
        scratch_shapes=[
            pltpu.VMEM((B, SQ, D), jnp.bfloat16),
            pltpu.VMEM((1, SQ, CHALF), jnp.bfloat16),
            pltpu.VMEM((1, SQ, CHALF), jnp.bfloat16),
            pltpu.VMEM((1, SQ, CHALF), jnp.bfloat16),
            pltpu.VMEM((1, SQ, CHALF), jnp.bfloat16),
            pltpu.SemaphoreType.DMA((6,)),
            pltpu.SemaphoreType.DMA((6,)),
        ],
        compiler_params=pltpu.CompilerParams(collective_id=0),
    )(x, Wq, Wo, K_ext, V_ext)


# device time: 19148 ns/iter; 1.7438x vs baseline; 1.0116x over previous
import jax
import jax.numpy as jnp
from jax import lax
from jax.experimental import pallas as pl
from jax.experimental.pallas import tpu as pltpu

N_DEV = 4
B, SQ, SKV, D = 2, 128, 128, 512
H_HEADS, DH = 8, 64
SCALE = 0.125
CHALF = D // 2


def kernel(x, Wq, Wo, K_ext, V_ext):
    def body(x_ref, wq_ref, wo_ref, k_ref, v_ref, out_ref,
             acc_ref, attn_ref, recvA1, recvA2, recvB1, recvB2,
             send_sems, recv_sems):
        my = lax.axis_index("i")
        p1 = my ^ 1
        p2 = my ^ 3
        ha = (my ^ (my >> 1)) & 1
        hb = (my >> 1) & 1

        wqb = wq_ref[...].astype(jnp.bfloat16)
        wob = wo_ref[...].astype(jnp.bfloat16)

        def compute_partial(b):
            xb = x_ref[pl.ds(b, 1)].reshape(SQ, D).astype(jnp.bfloat16)
            qb = lax.dot_general(
                xb, wqb, (((1,), (0,)), ((), ())),
                preferred_element_type=jnp.float32)
            qb = qb.astype(jnp.bfloat16)
            for h in range(H_HEADS):
                qh = qb[:, h * DH:(h + 1) * DH]
                kh = k_ref[pl.ds(b, 1), :, h, :].reshape(
                    SKV, DH).astype(jnp.bfloat16)
                vh = v_ref[pl.ds(b, 1), :, h, :].reshape(
                    SKV, DH).astype(jnp.bfloat16)
                s = lax.dot_general(
                    qh, kh, (((1,), (1,)), ((), ())),
                    preferred_element_type=jnp.float32) * SCALE
                m = jnp.max(s, axis=1, keepdims=True)
                p = jnp.exp(s - m)
                l = jnp.sum(p, axis=1, keepdims=True)
                pv = lax.dot_general(
                    p.astype(jnp.bfloat16), vh, (((1,), (0,)), ((), ())),
                    preferred_element_type=jnp.float32)
                attn_ref[:, h * DH:(h + 1) * DH] = (
                    pv * (1.0 / l)).astype(jnp.bfloat16)
            return lax.dot_general(
                attn_ref[...], wob, (((1,), (0,)), ((), ())),
                preferred_element_type=jnp.float32,
            ).astype(jnp.bfloat16).reshape(1, SQ, D)

        CA = pl.ds(0, CHALF)
        CB = pl.ds(CHALF, CHALF)

        def exchange(b, cols, dst, sem, dev):
            return pltpu.make_async_remote_copy(
                src_ref=acc_ref.at[pl.ds(b, 1), :, cols],
                dst_ref=dst,
                send_sem=send_sems.at[sem], recv_sem=recv_sems.at[sem],
                device_id=(dev,), device_id_type=pl.DeviceIdType.MESH,
            )

        def add(b, cols, recv):
            acc_ref[pl.ds(b, 1), :, cols] = (
                acc_ref[pl.ds(b, 1), :, cols].astype(jnp.float32)
                + recv[...].astype(jnp.float32)
            ).astype(jnp.bfloat16)

        acc_ref[pl.ds(1 - ha, 1)] = compute_partial(1 - ha)
        barrier = pltpu.get_barrier_semaphore()
        for nbr in (p1, p2):
            pl.semaphore_signal(barrier, inc=1, device_id=(nbr,),
                                device_id_type=pl.DeviceIdType.MESH,)
        pl.semaphore_wait(barrier, 2)

        a1 = exchange(1 - ha, CA, recvA1, 0, p1)
        a1.start()
        acc_ref[pl.ds(ha, 1)] = compute_partial(ha)
        b1 = exchange(1 - hb, CB, recvB1, 1, p2)
        b1.start()

        a1.wait()
        add(ha, CA, recvA1)
        a2 = exchange(ha, CA, recvA2, 2, p2)
        a2.start()

        b1.wait()
        add(hb, CB, recvB1)
        b2 = exchange(hb, CB, recvB2, 3, p1)
        b2.start()

        def emit(b, cols):
            out_ref[pl.ds(b, 1), :, cols] = (
                acc_ref[pl.ds(b, 1), :, cols].astype(jnp.float32))

        a2.wait()
        add(ha, CA, recvA2)
        a3 = exchange(ha, CA, acc_ref.at[pl.ds(ha, 1), :, CA], 4, p1)
        a3.start()
        emit(ha, CA)

        b2.wait()
        add(hb, CB, recvB2)
        b3 = exchange(hb, CB, acc_ref.at[pl.ds(hb, 1), :, CB], 5, p2)
        b3.start()
        emit(hb, CB)

        a3.wait()
        emit(1 - ha, CA)
        b3.wait()
        emit(1 - hb, CB)

    return pl.pallas_call(
        body,
        out_shape=jax.ShapeDtypeStruct((B, SQ, D), jnp.float32),
        in_specs=[pl.BlockSpec(memory_space=pltpu.VMEM)] * 5,
        out_specs=pl.BlockSpec(memory_space=pltpu.VMEM),
        scratch_shapes=[
            pltpu.VMEM((B, SQ, D), jnp.bfloat16),
            pltpu.VMEM((SQ, D), jnp.bfloat16),
            pltpu.VMEM((1, SQ, CHALF), jnp.bfloat16),
            pltpu.VMEM((1, SQ, CHALF), jnp.bfloat16),
            pltpu.VMEM((1, SQ, CHALF), jnp.bfloat16),
            pltpu.VMEM((1, SQ, CHALF), jnp.bfloat16),
            pltpu.SemaphoreType.DMA((6,)),
            pltpu.SemaphoreType.DMA((6,)),
        ],
        compiler_params=pltpu.CompilerParams(collective_id=0),
    )(x, Wq, Wo, K_ext, V_ext)
